# Initial kernel scaffold; baseline (speedup 1.0000x reference)
import jax
import jax.numpy as jnp
from jax.experimental import pallas as pl


def kernel(x, edge_index, Wl0, bl0, Wr0, Wl1, bl1, Wr1):
    raise NotImplementedError("write your pallas kernel here")


# speedup vs baseline: 4.4185x; 4.4185x over previous
import jax
import jax.numpy as jnp
from jax import lax
from jax.experimental import pallas as pl
from jax.experimental.pallas import tpu as pltpu
from jax.experimental.pallas import tpu_sc as plsc

NC = 2
NS = 16
NW = NC * NS
L = 16
CHUNK = 128


def _make_sc_pass(npad, d, k_chunks):
    mesh = plsc.VectorSubcoreMesh(core_axis_name="c", subcore_axis_name="s",
                                  num_cores=NC, num_subcores=NS)
    rpt = npad // NS

    def body(h_hbm, src_hbm, dst_hbm, psum_hbm, src_v, dst_v, rows_v, acc_sh,
             sem):
        c = lax.axis_index("c")
        s = lax.axis_index("s")
        wid = s * NC + c

        zeros16 = jnp.zeros((L,), jnp.float32)

        def zrow(i, carry):
            for t in range(d // L):
                rows_v[i, pl.ds(t * L, L)] = zeros16
            return carry

        lax.fori_loop(0, CHUNK, zrow, 0)
        for r in range(rpt // CHUNK):
            pltpu.sync_copy(rows_v, acc_sh.at[pl.ds(s * rpt + r * CHUNK, CHUNK)])

        pltpu.sync_copy(src_hbm.at[wid], src_v)
        pltpu.sync_copy(dst_hbm.at[wid], dst_v)

        plsc.subcore_barrier()

        def step(j, carry):
            pltpu.async_copy(h_hbm.at[src_v.at[j]], rows_v, sem).wait()
            pltpu.sync_copy(rows_v, acc_sh.at[dst_v.at[j]], add=True)
            return carry

        lax.fori_loop(0, k_chunks, step, 0)

        plsc.subcore_barrier()

        pltpu.sync_copy(acc_sh.at[pl.ds(s * rpt, rpt)],
                        psum_hbm.at[c, pl.ds(s * rpt, rpt)])

    return pl.kernel(
        body,
        out_type=jax.ShapeDtypeStruct((NC, npad, d), jnp.float32),
        mesh=mesh,
        scratch_types=(
            pltpu.VMEM((k_chunks, CHUNK), jnp.int32),
            pltpu.VMEM((k_chunks, CHUNK), jnp.int32),
            pltpu.VMEM((CHUNK, d), jnp.float32),
            pltpu.VMEM_SHARED((npad, d), jnp.float32),
            pltpu.SemaphoreType.DMA,
        ),
    )


def _make_sc_count_pass(npad, k_chunks):
    mesh = plsc.VectorSubcoreMesh(core_axis_name="c", subcore_axis_name="s",
                                  num_cores=NC, num_subcores=NS)
    rpt = npad // NS

    def body(dst_hbm, cnt_hbm, dst_v, ones_v, zero_v, cnt_sh):
        c = lax.axis_index("c")
        s = lax.axis_index("s")
        wid = s * NC + c

        def fill(i, carry):
            ones_v[i, pl.ds(0, L)] = jnp.full((L,), 1.0, jnp.float32)
            zero_v[i, pl.ds(0, L)] = jnp.zeros((L,), jnp.float32)
            return carry

        lax.fori_loop(0, CHUNK, fill, 0)
        for r in range(rpt // CHUNK):
            pltpu.sync_copy(zero_v, cnt_sh.at[pl.ds(s * rpt + r * CHUNK, CHUNK)])
        pltpu.sync_copy(dst_hbm.at[wid], dst_v)

        plsc.subcore_barrier()

        def step(j, carry):
            pltpu.sync_copy(ones_v, cnt_sh.at[dst_v.at[j]], add=True)
            return carry

        lax.fori_loop(0, k_chunks, step, 0)

        plsc.subcore_barrier()

        pltpu.sync_copy(cnt_sh.at[pl.ds(s * rpt, rpt)],
                        cnt_hbm.at[c, pl.ds(s * rpt, rpt)])

    return pl.kernel(
        body,
        out_type=jax.ShapeDtypeStruct((NC, npad, L), jnp.float32),
        mesh=mesh,
        compiler_params=pltpu.CompilerParams(use_tc_tiling_on_sc=False),
        scratch_types=(
            pltpu.VMEM((k_chunks, CHUNK), jnp.int32),
            pltpu.VMEM((CHUNK, L), jnp.float32),
            pltpu.VMEM((CHUNK, L), jnp.float32),
            pltpu.VMEM_SHARED((npad, L), jnp.float32),
        ),
    )


def _make_tc_layer1(npad, d, blk):
    def body(p_ref, c_ref, h_ref, wl_ref, bl_ref, wr_ref, out_ref, cnt_ref):
        ssum = p_ref[0] + p_ref[1]
        cnt = jnp.maximum((c_ref[0] + c_ref[1])[:, 0:1], 1.0)
        agg = ssum / cnt
        out = (jnp.dot(agg, wl_ref[...], preferred_element_type=jnp.float32)
               + bl_ref[...]
               + jnp.dot(h_ref[...], wr_ref[...],
                         preferred_element_type=jnp.float32))
        out_ref[...] = jnp.maximum(out, 0.0)
        cnt_ref[...] = cnt

    return pl.pallas_call(
        body,
        grid=(npad // blk,),
        in_specs=[
            pl.BlockSpec((NC, blk, d), lambda i: (0, i, 0)),
            pl.BlockSpec((NC, blk, L), lambda i: (0, i, 0)),
            pl.BlockSpec((blk, d), lambda i: (i, 0)),
            pl.BlockSpec((d, d), lambda i: (0, 0)),
            pl.BlockSpec((1, d), lambda i: (0, 0)),
            pl.BlockSpec((d, d), lambda i: (0, 0)),
        ],
        out_specs=[
            pl.BlockSpec((blk, d), lambda i: (i, 0)),
            pl.BlockSpec((blk, 1), lambda i: (i, 0)),
        ],
        out_shape=[
            jax.ShapeDtypeStruct((npad, d), jnp.float32),
            jax.ShapeDtypeStruct((npad, 1), jnp.float32),
        ],
    )


def _make_tc_layer2(npad, d, blk):
    def body(p_ref, cnt_ref, h_ref, wl_ref, bl_ref, wr_ref, out_ref):
        ssum = p_ref[0] + p_ref[1]
        agg = ssum / cnt_ref[...]
        out_ref[...] = (
            jnp.dot(agg, wl_ref[...], preferred_element_type=jnp.float32)
            + bl_ref[...]
            + jnp.dot(h_ref[...], wr_ref[...],
                      preferred_element_type=jnp.float32))

    return pl.pallas_call(
        body,
        grid=(npad // blk,),
        in_specs=[
            pl.BlockSpec((NC, blk, d), lambda i: (0, i, 0)),
            pl.BlockSpec((blk, 1), lambda i: (i, 0)),
            pl.BlockSpec((blk, d), lambda i: (i, 0)),
            pl.BlockSpec((d, d), lambda i: (0, 0)),
            pl.BlockSpec((1, d), lambda i: (0, 0)),
            pl.BlockSpec((d, d), lambda i: (0, 0)),
        ],
        out_specs=pl.BlockSpec((blk, d), lambda i: (i, 0)),
        out_shape=jax.ShapeDtypeStruct((npad, d), jnp.float32),
    )


@jax.jit
def kernel(x, edge_index, Wl0, bl0, Wr0, Wl1, bl1, Wr1):
    n, d = x.shape
    e = edge_index.shape[1]

    npad = (n // (NS * CHUNK) + 1) * (NS * CHUNK)
    k_chunks = -(-e // (CHUNK * NW))
    epad = k_chunks * CHUNK * NW

    src3 = jnp.pad(edge_index[0], (0, epad - e)).reshape(NW, k_chunks, CHUNK)
    dst3 = jnp.pad(edge_index[1], (0, epad - e),
                   constant_values=npad - 1).reshape(NW, k_chunks, CHUNK)
    xp = jnp.pad(x, ((0, npad - n), (0, 0)))

    blk = 512
    sc_feat = _make_sc_pass(npad, d, k_chunks)
    cntp = _make_sc_count_pass(npad, k_chunks)(dst3)
    psum0 = sc_feat(xp, src3, dst3)
    h1, cnt = _make_tc_layer1(npad, d, blk)(
        psum0, cntp, xp, Wl0, bl0[None, :], Wr0)
    psum1 = sc_feat(h1, src3, dst3)
    out = _make_tc_layer2(npad, d, blk)(
        psum1, cnt, h1, Wl1, bl1[None, :], Wr1)
    return out[:n]
